# baseline (speedup 1.0000x reference)
import functools

import jax
import jax.numpy as jnp
from jax import lax
from jax.experimental import pallas as pl
from jax.experimental.pallas import tpu as pltpu
from jax.experimental.pallas import tpu_sc as plsc

B, N, D = 4, 8192, 1024
R = B * N
NC, NS = 2, 16
NW = NC * NS
TROWS_PER_W = N // NW
C = 16
CD = C * D


NXB = 4
NTB = 2
NCHUNK = TROWS_PER_W // C
T_ITERS = NCHUNK * B
STEP = 8


def _sc_add(x2, tab):
    mesh = plsc.VectorSubcoreMesh(core_axis_name="c", subcore_axis_name="s")

    @functools.partial(
        pl.kernel,
        out_type=jax.ShapeDtypeStruct((R, D), jnp.float32),
        mesh=mesh,
        scratch_types=(
            [pltpu.VMEM((C, D), jnp.float32)] * (NXB + NTB)
            + [pltpu.SemaphoreType.DMA] * (2 * NXB + NTB)
        ),
    )
    def k(x_hbm, tab_hbm, out_hbm, *scratch):
        x_v = scratch[:NXB]
        tab_v = scratch[NXB:NXB + NTB]
        xin = scratch[NXB + NTB:2 * NXB + NTB]
        xout = scratch[2 * NXB + NTB:3 * NXB + NTB]
        tsem = scratch[3 * NXB + NTB:]
        wid = lax.axis_index("s") * NC + lax.axis_index("c")
        t_row = wid * TROWS_PER_W

        def tab_row(ci):
            return t_row + ci * C

        def x_row(u):
            return (u % B) * N + t_row + (u // B) * C

        def add_chunk(xb, tb):
            @plsc.parallel_loop(0, D, step=64, unroll=2)
            def _(c):
                for r in range(C):
                    for j in range(4):
                        sl = pl.ds(c + j * 16, 16)
                        plsc.addupdate(xb.at[r, sl], tb[r, sl])

        for q in range(NTB):
            pltpu.async_copy(tab_hbm.at[pl.ds(tab_row(q), C)], tab_v[q], tsem[q])
        for p in range(3):
            pltpu.async_copy(x_hbm.at[pl.ds(x_row(p), C)], x_v[p], xin[p])

        @pl.loop(0, T_ITERS, step=STEP)
        def _(t0):
            for kk in range(STEP):
                u = t0 + kk
                ci = t0 // B + kk // B
                p = kk % NXB
                q = (kk // B) % NTB
                if kk % B == 0:
                    pltpu.make_async_copy(
                        tab_hbm.at[pl.ds(tab_row(0), C)], tab_v[q], tsem[q]
                    ).wait()
                u2 = u + 3
                p2 = (kk + 3) % NXB

                @pl.when((u2 >= NXB) & (u2 < T_ITERS))
                def _():
                    pltpu.make_async_copy(
                        x_v[p2], out_hbm.at[pl.ds(x_row(0), C)], xout[p2]
                    ).wait()

                @pl.when(u2 < T_ITERS)
                def _():
                    pltpu.async_copy(
                        x_hbm.at[pl.ds(x_row(u2), C)], x_v[p2], xin[p2]
                    )

                pltpu.make_async_copy(
                    x_hbm.at[pl.ds(x_row(0), C)], x_v[p], xin[p]
                ).wait()
                add_chunk(x_v[p], tab_v[q])
                pltpu.async_copy(x_v[p], out_hbm.at[pl.ds(x_row(u), C)], xout[p])
                if kk % B == B - 1:
                    ci2 = ci + NTB

                    @pl.when(ci2 < NCHUNK)
                    def _():
                        pltpu.async_copy(
                            tab_hbm.at[pl.ds(tab_row(ci2), C)], tab_v[q], tsem[q]
                        )

        for p in range(NXB):
            pltpu.make_async_copy(
                x_v[p], out_hbm.at[pl.ds(x_row(0), C)], xout[p]
            ).wait()

    return k(x2, tab)


def _tc_body(x_ref, p_ref, o_ref):
    o_ref[...] = x_ref[...] + p_ref[...]


def _tc_add(x, pos_table):
    BN = 2048
    grid = (N // BN, B)
    return pl.pallas_call(
        _tc_body,
        grid=grid,
        in_specs=[
            pl.BlockSpec((1, BN, D), lambda i, b: (b, i, 0)),
            pl.BlockSpec((BN, D), lambda i, b: (i, 0)),
        ],
        out_specs=pl.BlockSpec((1, BN, D), lambda i, b: (b, i, 0)),
        out_shape=jax.ShapeDtypeStruct((B, N, D), x.dtype),
    )(x, pos_table[:N])


def kernel(x, pos_table):
    out2 = _sc_add(x.reshape(R, D), pos_table[:N])
    return out2.reshape(B, N, D)


# speedup vs baseline: 1.3225x; 1.3225x over previous
import functools

import jax
import jax.numpy as jnp
from jax import lax
from jax.experimental import pallas as pl
from jax.experimental.pallas import tpu as pltpu
from jax.experimental.pallas import tpu_sc as plsc

B, N, D = 4, 8192, 1024
R = B * N
NC, NS = 2, 16
NW = NC * NS
TROWS_PER_W = N // NW
C = 16
CD = C * D


NXB = 4
NTB = 2
NCHUNK = TROWS_PER_W // C
B_SC = 1
B0 = B - B_SC
T_ITERS = NCHUNK * B_SC
STEP = 8


def _sc_add(x2, tab):
    mesh = plsc.VectorSubcoreMesh(core_axis_name="c", subcore_axis_name="s")

    @functools.partial(
        pl.kernel,
        out_type=jax.ShapeDtypeStruct((R, D), jnp.float32),
        mesh=mesh,
        scratch_types=(
            [pltpu.VMEM((C, D), jnp.float32)] * (NXB + NTB)
            + [pltpu.SemaphoreType.DMA] * (2 * NXB + NTB)
        ),
    )
    def k(x_hbm, tab_hbm, out_hbm, *scratch):
        x_v = scratch[:NXB]
        tab_v = scratch[NXB:NXB + NTB]
        xin = scratch[NXB + NTB:2 * NXB + NTB]
        xout = scratch[2 * NXB + NTB:3 * NXB + NTB]
        tsem = scratch[3 * NXB + NTB:]
        wid = lax.axis_index("s") * NC + lax.axis_index("c")
        t_row = wid * TROWS_PER_W

        def tab_row(ci):
            return t_row + ci * C

        def x_row(u):
            return (B0 + u % B_SC) * N + t_row + (u // B_SC) * C

        def add_chunk(xb, tb):
            @plsc.parallel_loop(0, D, step=64, unroll=2)
            def _(c):
                for r in range(C):
                    for j in range(4):
                        sl = pl.ds(c + j * 16, 16)
                        plsc.addupdate(xb.at[r, sl], tb[r, sl])

        for q in range(NTB):
            pltpu.async_copy(tab_hbm.at[pl.ds(tab_row(q), C)], tab_v[q], tsem[q])
        for p in range(2):
            pltpu.async_copy(x_hbm.at[pl.ds(x_row(p), C)], x_v[p], xin[p])

        @pl.loop(0, T_ITERS, step=STEP)
        def _(t0):
            for kk in range(STEP):
                u = t0 + kk
                ci = t0 // B_SC + kk // B_SC
                p = kk % NXB
                q = (kk // B_SC) % NTB
                if kk % B_SC == 0:
                    pltpu.make_async_copy(
                        tab_hbm.at[pl.ds(tab_row(0), C)], tab_v[q], tsem[q]
                    ).wait()
                u2 = u + 2
                p2 = (kk + 2) % NXB

                @pl.when((u2 >= NXB) & (u2 < T_ITERS))
                def _():
                    pltpu.make_async_copy(
                        x_v[p2], out_hbm.at[pl.ds(x_row(0), C)], xout[p2]
                    ).wait()

                @pl.when(u2 < T_ITERS)
                def _():
                    pltpu.async_copy(
                        x_hbm.at[pl.ds(x_row(u2), C)], x_v[p2], xin[p2]
                    )

                pltpu.make_async_copy(
                    x_hbm.at[pl.ds(x_row(0), C)], x_v[p], xin[p]
                ).wait()
                add_chunk(x_v[p], tab_v[q])
                pltpu.async_copy(x_v[p], out_hbm.at[pl.ds(x_row(u), C)], xout[p])
                if kk % B_SC == B_SC - 1:
                    ci2 = ci + NTB

                    @pl.when(ci2 < NCHUNK)
                    def _():
                        pltpu.async_copy(
                            tab_hbm.at[pl.ds(tab_row(ci2), C)], tab_v[q], tsem[q]
                        )

        for p in range(NXB):
            pltpu.make_async_copy(
                x_v[p], out_hbm.at[pl.ds(x_row(0), C)], xout[p]
            ).wait()

    return k(x2, tab)


def _tc_body(a_ref, x_ref, p_ref, o_ref):
    o_ref[...] = x_ref[...] + p_ref[...]


def _tc_fill(sc_out, x, tab):
    BN = 2048
    grid = (N // BN, B0)
    return pl.pallas_call(
        _tc_body,
        grid=grid,
        in_specs=[
            pl.BlockSpec((1, BN, D), lambda i, b: (B - 1, 0, 0)),
            pl.BlockSpec((1, BN, D), lambda i, b: (b, i, 0)),
            pl.BlockSpec((BN, D), lambda i, b: (i, 0)),
        ],
        out_specs=pl.BlockSpec((1, BN, D), lambda i, b: (b, i, 0)),
        out_shape=jax.ShapeDtypeStruct((B, N, D), x.dtype),
        input_output_aliases={0: 0},
    )(sc_out, x, tab)


def kernel(x, pos_table):
    tab = pos_table[:N]
    sc_out = _sc_add(x.reshape(R, D), tab)
    return _tc_fill(sc_out.reshape(B, N, D), x, tab)


# speedup vs baseline: 1.3518x; 1.0221x over previous
import functools

import jax
import jax.numpy as jnp
from jax import lax
from jax.experimental import pallas as pl
from jax.experimental.pallas import tpu as pltpu
from jax.experimental.pallas import tpu_sc as plsc

B, N, D = 4, 8192, 1024
R = B * N
NC, NS = 2, 16
NW = NC * NS
TROWS_PER_W = N // NW
C = 16
CD = C * D


NXB = 4
NTB = 2
NCHUNK = TROWS_PER_W // C
B_SC = 1
B0 = B - B_SC
T_ITERS = NCHUNK * B_SC
STEP = 8


def _sc_add(x2, tab):
    mesh = plsc.VectorSubcoreMesh(core_axis_name="c", subcore_axis_name="s")

    @functools.partial(
        pl.kernel,
        out_type=jax.ShapeDtypeStruct((R, D), jnp.float32),
        mesh=mesh,
        scratch_types=(
            [pltpu.VMEM((C, D), jnp.float32)] * (NXB + NTB)
            + [pltpu.SemaphoreType.DMA] * (2 * NXB + NTB)
        ),
    )
    def k(x_hbm, tab_hbm, out_hbm, *scratch):
        x_v = scratch[:NXB]
        tab_v = scratch[NXB:NXB + NTB]
        xin = scratch[NXB + NTB:2 * NXB + NTB]
        xout = scratch[2 * NXB + NTB:3 * NXB + NTB]
        tsem = scratch[3 * NXB + NTB:]
        wid = lax.axis_index("s") * NC + lax.axis_index("c")
        t_row = wid * TROWS_PER_W

        def tab_row(ci):
            return t_row + ci * C

        def x_row(u):
            return (B0 + u % B_SC) * N + t_row + (u // B_SC) * C

        def add_chunk(xb, tb):
            @plsc.parallel_loop(0, D, step=64, unroll=2)
            def _(c):
                for r in range(C):
                    for j in range(4):
                        sl = pl.ds(c + j * 16, 16)
                        plsc.addupdate(xb.at[r, sl], tb[r, sl])

        for q in range(NTB):
            pltpu.async_copy(tab_hbm.at[pl.ds(tab_row(q), C)], tab_v[q], tsem[q])
        for p in range(2):
            pltpu.async_copy(x_hbm.at[pl.ds(x_row(p), C)], x_v[p], xin[p])

        @pl.loop(0, T_ITERS, step=STEP)
        def _(t0):
            for kk in range(STEP):
                u = t0 + kk
                ci = t0 // B_SC + kk // B_SC
                p = kk % NXB
                q = (kk // B_SC) % NTB
                if kk % B_SC == 0:
                    pltpu.make_async_copy(
                        tab_hbm.at[pl.ds(tab_row(0), C)], tab_v[q], tsem[q]
                    ).wait()
                u2 = u + 2
                p2 = (kk + 2) % NXB

                @pl.when((u2 >= NXB) & (u2 < T_ITERS))
                def _():
                    pltpu.make_async_copy(
                        x_v[p2], out_hbm.at[pl.ds(x_row(0), C)], xout[p2]
                    ).wait()

                @pl.when(u2 < T_ITERS)
                def _():
                    pltpu.async_copy(
                        x_hbm.at[pl.ds(x_row(u2), C)], x_v[p2], xin[p2]
                    )

                pltpu.make_async_copy(
                    x_hbm.at[pl.ds(x_row(0), C)], x_v[p], xin[p]
                ).wait()
                add_chunk(x_v[p], tab_v[q])
                pltpu.async_copy(x_v[p], out_hbm.at[pl.ds(x_row(u), C)], xout[p])
                if kk % B_SC == B_SC - 1:
                    ci2 = ci + NTB

                    @pl.when(ci2 < NCHUNK)
                    def _():
                        pltpu.async_copy(
                            tab_hbm.at[pl.ds(tab_row(ci2), C)], tab_v[q], tsem[q]
                        )

        for p in range(NXB):
            pltpu.make_async_copy(
                x_v[p], out_hbm.at[pl.ds(x_row(0), C)], xout[p]
            ).wait()

    return k(x2, tab)


def _tc_body(a_ref, x_ref, p_ref, o_ref):
    o_ref[...] = x_ref[...] + p_ref[...]


def _tc_fill(sc_out, x, tab):
    BN = 2048
    grid = (N // BN, B0)
    return pl.pallas_call(
        _tc_body,
        grid=grid,
        in_specs=[
            pl.BlockSpec((1, 8, D), lambda i, b: (B - 1, 0, 0)),
            pl.BlockSpec((1, BN, D), lambda i, b: (b, i, 0)),
            pl.BlockSpec((BN, D), lambda i, b: (i, 0)),
        ],
        out_specs=pl.BlockSpec((1, BN, D), lambda i, b: (b, i, 0)),
        out_shape=jax.ShapeDtypeStruct((B, N, D), x.dtype),
        input_output_aliases={0: 0},
    )(sc_out, x, tab)


def kernel(x, pos_table):
    tab = pos_table[:N]
    sc_out = _sc_add(x.reshape(R, D), tab)
    return _tc_fill(sc_out.reshape(B, N, D), x, tab)
